# Initial kernel scaffold; baseline (speedup 1.0000x reference)
import functools

import jax
import jax.numpy as jnp
from jax import lax
from jax.experimental import pallas as pl
from jax.experimental.pallas import tpu as pltpu
from jax.experimental.pallas import tpu_sc as plsc

N = 10000
E = 320000
D = 128
C = 40

NC = 2
NS = 16
NTILES = NC * NS
B = 128
NB = -(-E // (NTILES * B))
EPAD = NTILES * NB * B
NPAD = 10240
RPS = NPAD // NS

_mesh = plsc.VectorSubcoreMesh(core_axis_name="c", subcore_axis_name="s")


def _wait_copy(src, dst, sem):
    pltpu.make_async_copy(src, dst, sem).wait()


@functools.partial(
    pl.kernel,
    out_type=jax.ShapeDtypeStruct((NC, NPAD, 1), jnp.float32),
    mesh=_mesh,
    scratch_types=[
        pltpu.VMEM((NB, B), jnp.int32),
        pltpu.VMEM((B, 1), jnp.float32),
        pltpu.VMEM_SHARED((NPAD, 1), jnp.float32),
    ],
)
def _deg_kernel(dst_hbm, ones_hbm, zeros_hbm, deg_out, dstv, onesv, acc):
    c = lax.axis_index("c")
    s = lax.axis_index("s")
    wid = c * NS + s
    pltpu.sync_copy(dst_hbm.at[wid], dstv)
    pltpu.sync_copy(ones_hbm, onesv)
    pltpu.sync_copy(zeros_hbm.at[pl.ds(s * RPS, RPS)], acc.at[pl.ds(s * RPS, RPS)])
    plsc.subcore_barrier()

    def step(j, carry):
        pltpu.sync_copy(onesv, acc.at[dstv.at[j]], add=True)
        return carry

    lax.fori_loop(0, NB, step, 0)
    plsc.subcore_barrier()
    pltpu.sync_copy(acc.at[pl.ds(s * RPS, RPS)],
                    deg_out.at[c, pl.ds(s * RPS, RPS)])


@functools.partial(
    pl.kernel,
    out_type=jax.ShapeDtypeStruct((NC, NPAD, 16), jnp.float32),
    mesh=_mesh,
    scratch_types=[
        pltpu.VMEM((NB, B), jnp.int32),
        pltpu.VMEM((NB, B), jnp.int32),
        pltpu.VMEM((B, 16), jnp.float32),
        pltpu.VMEM((B, 16), jnp.float32),
        pltpu.VMEM_SHARED((NPAD, 16), jnp.float32),
        pltpu.SemaphoreType.DMA,
        pltpu.SemaphoreType.DMA,
    ],
)
def _prop_kernel(g_hbm, src_hbm, dst_hbm, zeros_hbm, out_hbm,
                 srcv, dstv, bufa, bufb, acc, sema, semb):
    c = lax.axis_index("c")
    s = lax.axis_index("s")
    wid = c * NS + s
    pltpu.sync_copy(src_hbm.at[wid], srcv)
    pltpu.sync_copy(dst_hbm.at[wid], dstv)
    pltpu.sync_copy(zeros_hbm.at[pl.ds(s * RPS, RPS)], acc.at[pl.ds(s * RPS, RPS)])
    plsc.subcore_barrier()

    pltpu.async_copy(g_hbm.at[srcv.at[0]], bufa, sema)

    def pair(i, carry):
        j = 2 * i
        _wait_copy(g_hbm.at[pl.ds(0, B)], bufa, sema)
        pltpu.async_copy(g_hbm.at[srcv.at[j + 1]], bufb, semb)
        pltpu.sync_copy(bufa, acc.at[dstv.at[j]], add=True)
        _wait_copy(g_hbm.at[pl.ds(0, B)], bufb, semb)
        pltpu.async_copy(g_hbm.at[srcv.at[j + 2]], bufa, sema)
        pltpu.sync_copy(bufb, acc.at[dstv.at[j + 1]], add=True)
        return carry

    lax.fori_loop(0, (NB - 1) // 2, pair, 0)
    _wait_copy(g_hbm.at[pl.ds(0, B)], bufa, sema)
    pltpu.sync_copy(bufa, acc.at[dstv.at[NB - 1]], add=True)

    plsc.subcore_barrier()
    pltpu.sync_copy(acc.at[pl.ds(s * RPS, RPS)],
                    out_hbm.at[c, pl.ds(s * RPS, RPS)])


def _tc1_body(x_ref, w_ref, degp_ref, g_ref, dinv_ref):
    deg = degp_ref[0] + degp_ref[1] + 1.0
    dinv = lax.rsqrt(deg)
    dinv_ref[...] = dinv
    h = jnp.dot(x_ref[...], w_ref[...], preferred_element_type=jnp.float32)
    g_ref[0:N] = h * dinv[0:N]
    g_ref[N:NPAD] = jnp.zeros((NPAD - N, 16), jnp.float32)


_tc1 = pl.pallas_call(
    _tc1_body,
    out_shape=(jax.ShapeDtypeStruct((NPAD, 16), jnp.float32),
               jax.ShapeDtypeStruct((NPAD, 1), jnp.float32)),
)


def _tc_mid_body(sp_ref, g_ref, dinv_ref, b_ref, w_ref, gout_ref):
    dinv = dinv_ref[...]
    h = dinv * (sp_ref[0] + sp_ref[1] + g_ref[...]) + b_ref[...]
    h = jnp.maximum(h, 0.0)
    gout_ref[...] = jnp.dot(h, w_ref[...], preferred_element_type=jnp.float32) * dinv


_tc_mid = pl.pallas_call(
    _tc_mid_body,
    out_shape=jax.ShapeDtypeStruct((NPAD, 16), jnp.float32),
)


def _tc_head_body(sp_ref, g_ref, dinv_ref, b3_ref, m1_ref, mb1_ref, g1_ref,
                  be1_ref, m2_ref, mb2_ref, g2_ref, be2_ref, m3_ref, mb3_ref,
                  out_ref):
    h = dinv_ref[...] * (sp_ref[0] + sp_ref[1] + g_ref[...]) + b3_ref[...]
    mask = (lax.broadcasted_iota(jnp.int32, (NPAD, 1), 0) < N).astype(jnp.float32)

    def bn(t, gamma, beta):
        mu = jnp.sum(t * mask, axis=0, keepdims=True) * (1.0 / N)
        d = (t - mu) * mask
        var = jnp.sum(d * d, axis=0, keepdims=True) * (1.0 / N)
        return (t - mu) * lax.rsqrt(var + 1e-5) * gamma + beta

    def leaky(t):
        return jnp.where(t > 0, t, 0.02 * t)

    t = jnp.dot(h, m1_ref[...], preferred_element_type=jnp.float32) + mb1_ref[...]
    t = leaky(bn(t, g1_ref[...], be1_ref[...]))
    t = jnp.dot(t, m2_ref[...], preferred_element_type=jnp.float32) + mb2_ref[...]
    t = leaky(bn(t, g2_ref[...], be2_ref[...]))
    t = jnp.dot(t, m3_ref[...], preferred_element_type=jnp.float32) + mb3_ref[...]
    m = jnp.max(t, axis=1, keepdims=True)
    lse = jnp.log(jnp.sum(jnp.exp(t - m), axis=1, keepdims=True))
    out_ref[...] = t - m - lse


_tc_head = pl.pallas_call(
    _tc_head_body,
    out_shape=jax.ShapeDtypeStruct((NPAD, C), jnp.float32),
)


def kernel(x, edge_index, W1, b1, W2, b2, W3, b3,
           M1, mb1, g1, be1, M2, mb2, g2, be2, M3, mb3):
    x = jnp.squeeze(x)
    src, dst = edge_index[0], edge_index[1]
    pad = EPAD - E
    srcp = jnp.concatenate([src, jnp.zeros((pad,), jnp.int32)]).reshape(NTILES, NB, B)
    dstp = jnp.concatenate([dst, jnp.full((pad,), N, jnp.int32)]).reshape(NTILES, NB, B)

    ones_col = jnp.ones((B, 1), jnp.float32)
    zeros_col = jnp.zeros((NPAD, 1), jnp.float32)
    zeros_rows = jnp.zeros((NPAD, 16), jnp.float32)

    degp = _deg_kernel(dstp, ones_col, zeros_col)
    gv, dinv = _tc1(x, W1, degp)
    sp = _prop_kernel(gv, srcp, dstp, zeros_rows)
    gv = _tc_mid(sp, gv, dinv, b1, W2)
    sp = _prop_kernel(gv, srcp, dstp, zeros_rows)
    gv = _tc_mid(sp, gv, dinv, b2, W3)
    sp = _prop_kernel(gv, srcp, dstp, zeros_rows)
    out = _tc_head(sp, gv, dinv, b3, M1, mb1, g1, be1, M2, mb2, g2, be2, M3, mb3)
    return out[:N]


# speedup vs baseline: 30.2606x; 30.2606x over previous
import functools

import jax
import jax.numpy as jnp
from jax import lax
from jax.experimental import pallas as pl
from jax.experimental.pallas import tpu as pltpu
from jax.experimental.pallas import tpu_sc as plsc

N = 10000
E = 320000
D = 128
C = 40

NC = 2
NS = 16
NTILES = NC * NS
B = 128
NB = -(-E // (NTILES * B))
EPAD = NTILES * NB * B
NPAD = 10240
RPS = NPAD // NS

def _wait_copy(src, dst, sem):
    pltpu.make_async_copy(src, dst, sem).wait()


def _deg_body(dst_hbm, ones_hbm, zeros_hbm, deg_out, dstv, onesv, acc):
    c = lax.axis_index("c")
    s = lax.axis_index("s")
    wid = c * NS + s
    pltpu.sync_copy(dst_hbm.at[wid], dstv)
    pltpu.sync_copy(ones_hbm, onesv)
    pltpu.sync_copy(zeros_hbm.at[pl.ds(s * RPS, RPS)], acc.at[pl.ds(s * RPS, RPS)])
    plsc.subcore_barrier()

    def step(j, carry):
        pltpu.sync_copy(onesv, acc.at[dstv.at[j]], add=True)
        return carry

    lax.fori_loop(0, NB, step, 0)
    plsc.subcore_barrier()
    pltpu.sync_copy(acc.at[pl.ds(s * RPS, RPS)],
                    deg_out.at[c, pl.ds(s * RPS, RPS)])


def _prop_body(g_hbm, src_hbm, dst_hbm, zeros_hbm, out_hbm,
               srcv, dstv, bufa, bufb, acc, sema, semb):
    c = lax.axis_index("c")
    s = lax.axis_index("s")
    wid = c * NS + s
    pltpu.sync_copy(src_hbm.at[wid], srcv)
    pltpu.sync_copy(dst_hbm.at[wid], dstv)
    pltpu.sync_copy(zeros_hbm.at[pl.ds(s * RPS, RPS)], acc.at[pl.ds(s * RPS, RPS)])
    plsc.subcore_barrier()

    pltpu.async_copy(g_hbm.at[srcv.at[0]], bufa, sema)

    def pair(i, carry):
        j = 2 * i
        _wait_copy(g_hbm.at[pl.ds(0, B)], bufa, sema)
        pltpu.async_copy(g_hbm.at[srcv.at[j + 1]], bufb, semb)
        pltpu.sync_copy(bufa, acc.at[dstv.at[j]], add=True)
        _wait_copy(g_hbm.at[pl.ds(0, B)], bufb, semb)
        pltpu.async_copy(g_hbm.at[srcv.at[j + 2]], bufa, sema)
        pltpu.sync_copy(bufb, acc.at[dstv.at[j + 1]], add=True)
        return carry

    lax.fori_loop(0, (NB - 1) // 2, pair, 0)
    _wait_copy(g_hbm.at[pl.ds(0, B)], bufa, sema)
    pltpu.sync_copy(bufa, acc.at[dstv.at[NB - 1]], add=True)

    plsc.subcore_barrier()
    pltpu.sync_copy(acc.at[pl.ds(s * RPS, RPS)],
                    out_hbm.at[c, pl.ds(s * RPS, RPS)])


@functools.cache
def _sc_kernels():
    mesh = plsc.VectorSubcoreMesh(core_axis_name="c", subcore_axis_name="s",
                                  num_cores=NC, num_subcores=NS)
    params = pltpu.CompilerParams(use_tc_tiling_on_sc=False)
    deg = pl.kernel(
        _deg_body,
        out_type=jax.ShapeDtypeStruct((NC, NPAD, 1), jnp.float32),
        mesh=mesh,
        compiler_params=params,
        scratch_types=[
            pltpu.VMEM((NB, B), jnp.int32),
            pltpu.VMEM((B, 1), jnp.float32),
            pltpu.VMEM_SHARED((NPAD, 1), jnp.float32),
        ],
    )
    prop = pl.kernel(
        _prop_body,
        out_type=jax.ShapeDtypeStruct((NC, NPAD, 16), jnp.float32),
        mesh=mesh,
        compiler_params=params,
        scratch_types=[
            pltpu.VMEM((NB, B), jnp.int32),
            pltpu.VMEM((NB, B), jnp.int32),
            pltpu.VMEM((B, 16), jnp.float32),
            pltpu.VMEM((B, 16), jnp.float32),
            pltpu.VMEM_SHARED((NPAD, 16), jnp.float32),
            pltpu.SemaphoreType.DMA,
            pltpu.SemaphoreType.DMA,
        ],
    )
    return deg, prop


def _tc1_body(x_ref, w_ref, degp_ref, g_ref, dinv_ref):
    deg = degp_ref[0] + degp_ref[1] + 1.0
    dinv = lax.rsqrt(deg)
    dinv_ref[...] = dinv
    h = jnp.dot(x_ref[...], w_ref[...], preferred_element_type=jnp.float32)
    g_ref[0:N] = h * dinv[0:N]
    g_ref[N:NPAD] = jnp.zeros((NPAD - N, 16), jnp.float32)


_tc1 = pl.pallas_call(
    _tc1_body,
    out_shape=(jax.ShapeDtypeStruct((NPAD, 16), jnp.float32),
               jax.ShapeDtypeStruct((NPAD, 1), jnp.float32)),
)


def _tc_mid_body(sp_ref, g_ref, dinv_ref, b_ref, w_ref, gout_ref):
    dinv = dinv_ref[...]
    h = dinv * (sp_ref[0] + sp_ref[1] + g_ref[...]) + b_ref[...]
    h = jnp.maximum(h, 0.0)
    gout_ref[...] = jnp.dot(h, w_ref[...], preferred_element_type=jnp.float32) * dinv


_tc_mid = pl.pallas_call(
    _tc_mid_body,
    out_shape=jax.ShapeDtypeStruct((NPAD, 16), jnp.float32),
)


def _tc_head_body(sp_ref, g_ref, dinv_ref, b3_ref, m1_ref, mb1_ref, g1_ref,
                  be1_ref, m2_ref, mb2_ref, g2_ref, be2_ref, m3_ref, mb3_ref,
                  out_ref):
    h = dinv_ref[...] * (sp_ref[0] + sp_ref[1] + g_ref[...]) + b3_ref[...]
    mask = (lax.broadcasted_iota(jnp.int32, (NPAD, 1), 0) < N).astype(jnp.float32)

    def bn(t, gamma, beta):
        mu = jnp.sum(t * mask, axis=0, keepdims=True) * (1.0 / N)
        d = (t - mu) * mask
        var = jnp.sum(d * d, axis=0, keepdims=True) * (1.0 / N)
        return (t - mu) * lax.rsqrt(var + 1e-5) * gamma + beta

    def leaky(t):
        return jnp.where(t > 0, t, 0.02 * t)

    t = jnp.dot(h, m1_ref[...], preferred_element_type=jnp.float32) + mb1_ref[...]
    t = leaky(bn(t, g1_ref[...], be1_ref[...]))
    t = jnp.dot(t, m2_ref[...], preferred_element_type=jnp.float32) + mb2_ref[...]
    t = leaky(bn(t, g2_ref[...], be2_ref[...]))
    t = jnp.dot(t, m3_ref[...], preferred_element_type=jnp.float32) + mb3_ref[...]
    m = jnp.max(t, axis=1, keepdims=True)
    lse = jnp.log(jnp.sum(jnp.exp(t - m), axis=1, keepdims=True))
    out_ref[...] = t - m - lse


_tc_head = pl.pallas_call(
    _tc_head_body,
    out_shape=jax.ShapeDtypeStruct((NPAD, C), jnp.float32),
)


def kernel(x, edge_index, W1, b1, W2, b2, W3, b3,
           M1, mb1, g1, be1, M2, mb2, g2, be2, M3, mb3):
    x = jnp.squeeze(x)
    src, dst = edge_index[0], edge_index[1]
    pad = EPAD - E
    srcp = jnp.concatenate([src, jnp.zeros((pad,), jnp.int32)]).reshape(NTILES, NB, B)
    dstp = jnp.concatenate([dst, jnp.full((pad,), N, jnp.int32)]).reshape(NTILES, NB, B)

    ones_col = jnp.ones((B, 1), jnp.float32)
    zeros_col = jnp.zeros((NPAD, 1), jnp.float32)
    zeros_rows = jnp.zeros((NPAD, 16), jnp.float32)

    deg_kernel, prop_kernel = _sc_kernels()
    degp = deg_kernel(dstp, ones_col, zeros_col)
    gv, dinv = _tc1(x, W1, degp)
    sp = prop_kernel(gv, srcp, dstp, zeros_rows)
    gv = _tc_mid(sp, gv, dinv, b1, W2)
    sp = prop_kernel(gv, srcp, dstp, zeros_rows)
    gv = _tc_mid(sp, gv, dinv, b2, W3)
    sp = prop_kernel(gv, srcp, dstp, zeros_rows)
    out = _tc_head(sp, gv, dinv, b3, M1, mb1, g1, be1, M2, mb2, g2, be2, M3, mb3)
    return out[:N]


# speedup vs baseline: 34.6265x; 1.1443x over previous
import functools

import jax
import jax.numpy as jnp
from jax import lax
from jax.experimental import pallas as pl
from jax.experimental.pallas import tpu as pltpu
from jax.experimental.pallas import tpu_sc as plsc

N = 10000
E = 320000
D = 128
C = 40

NC = 2
NS = 16
NTILES = NC * NS
B = 128
NB = 80
EPAD = NTILES * NB * B
NSLOT = 8
NPAD = 10240
RPS = NPAD // NS

def _wait_copy(src, dst, sem):
    pltpu.make_async_copy(src, dst, sem).wait()


def _deg_body(dst_hbm, ones_hbm, zeros_hbm, deg_out, dstv, onesv, acc):
    c = lax.axis_index("c")
    s = lax.axis_index("s")
    wid = c * NS + s
    pltpu.sync_copy(dst_hbm.at[wid], dstv)
    pltpu.sync_copy(ones_hbm, onesv)
    pltpu.sync_copy(zeros_hbm.at[pl.ds(s * RPS, RPS)], acc.at[pl.ds(s * RPS, RPS)])
    plsc.subcore_barrier()

    def step(j, carry):
        pltpu.sync_copy(onesv, acc.at[dstv.at[j]], add=True)
        return carry

    lax.fori_loop(0, NB, step, 0)
    plsc.subcore_barrier()
    pltpu.sync_copy(acc.at[pl.ds(s * RPS, RPS)],
                    deg_out.at[c, pl.ds(s * RPS, RPS)])


def _prop_body(g_hbm, src_hbm, dst_hbm, zeros_hbm, out_hbm,
               srcv, dstv, bufs, acc, *gsems):
    c = lax.axis_index("c")
    s = lax.axis_index("s")
    wid = c * NS + s
    pltpu.sync_copy(src_hbm.at[wid], srcv)
    pltpu.sync_copy(dst_hbm.at[wid], dstv)
    pltpu.sync_copy(zeros_hbm.at[pl.ds(s * RPS, RPS)], acc.at[pl.ds(s * RPS, RPS)])
    plsc.subcore_barrier()

    def fire_gather(j, slot):
        pltpu.async_copy(g_hbm.at[srcv.at[j]], bufs[slot], gsems[slot])

    def wait_gather(slot):
        _wait_copy(g_hbm.at[srcv.at[0]], bufs[slot], gsems[slot])

    for j in range(NSLOT):
        fire_gather(j, j)

    def group(g, carry):
        base = g * NSLOT
        for b in range(NSLOT):
            i = base + b
            wait_gather(b)
            pltpu.sync_copy(bufs[b], acc.at[dstv.at[i]], add=True)
            fire_gather(i + NSLOT, b)
        return carry

    lax.fori_loop(0, NB // NSLOT - 1, group, 0)

    for i in range(NB - NSLOT, NB):
        b = i % NSLOT
        wait_gather(b)
        pltpu.sync_copy(bufs[b], acc.at[dstv.at[i]], add=True)

    plsc.subcore_barrier()
    pltpu.sync_copy(acc.at[pl.ds(s * RPS, RPS)],
                    out_hbm.at[c, pl.ds(s * RPS, RPS)])


@functools.cache
def _sc_kernels():
    mesh = plsc.VectorSubcoreMesh(core_axis_name="c", subcore_axis_name="s",
                                  num_cores=NC, num_subcores=NS)
    params = pltpu.CompilerParams(use_tc_tiling_on_sc=False)
    deg = pl.kernel(
        _deg_body,
        out_type=jax.ShapeDtypeStruct((NC, NPAD, 1), jnp.float32),
        mesh=mesh,
        compiler_params=params,
        scratch_types=[
            pltpu.VMEM((NB, B), jnp.int32),
            pltpu.VMEM((B, 1), jnp.float32),
            pltpu.VMEM_SHARED((NPAD, 1), jnp.float32),
        ],
    )
    prop = pl.kernel(
        _prop_body,
        out_type=jax.ShapeDtypeStruct((NC, NPAD, 16), jnp.float32),
        mesh=mesh,
        compiler_params=params,
        scratch_types=[
            pltpu.VMEM((NB, B), jnp.int32),
            pltpu.VMEM((NB, B), jnp.int32),
            tuple(pltpu.VMEM((B, 16), jnp.float32) for _ in range(NSLOT)),
            pltpu.VMEM_SHARED((NPAD, 16), jnp.float32),
        ] + [pltpu.SemaphoreType.DMA for _ in range(NSLOT)],
    )
    return deg, prop


def _tc1_body(x_ref, w_ref, degp_ref, g_ref, dinv_ref):
    deg = degp_ref[0] + degp_ref[1] + 1.0
    dinv = lax.rsqrt(deg)
    dinv_ref[...] = dinv
    h = jnp.dot(x_ref[...], w_ref[...], preferred_element_type=jnp.float32)
    g_ref[0:N] = h * dinv[0:N]
    g_ref[N:NPAD] = jnp.zeros((NPAD - N, 16), jnp.float32)


_tc1 = pl.pallas_call(
    _tc1_body,
    out_shape=(jax.ShapeDtypeStruct((NPAD, 16), jnp.float32),
               jax.ShapeDtypeStruct((NPAD, 1), jnp.float32)),
)


def _tc_mid_body(sp_ref, g_ref, dinv_ref, b_ref, w_ref, gout_ref):
    dinv = dinv_ref[...]
    h = dinv * (sp_ref[0] + sp_ref[1] + g_ref[...]) + b_ref[...]
    h = jnp.maximum(h, 0.0)
    gout_ref[...] = jnp.dot(h, w_ref[...], preferred_element_type=jnp.float32) * dinv


_tc_mid = pl.pallas_call(
    _tc_mid_body,
    out_shape=jax.ShapeDtypeStruct((NPAD, 16), jnp.float32),
)


def _tc_head_body(sp_ref, g_ref, dinv_ref, b3_ref, m1_ref, mb1_ref, g1_ref,
                  be1_ref, m2_ref, mb2_ref, g2_ref, be2_ref, m3_ref, mb3_ref,
                  out_ref):
    h = dinv_ref[...] * (sp_ref[0] + sp_ref[1] + g_ref[...]) + b3_ref[...]
    mask = (lax.broadcasted_iota(jnp.int32, (NPAD, 1), 0) < N).astype(jnp.float32)

    def bn(t, gamma, beta):
        mu = jnp.sum(t * mask, axis=0, keepdims=True) * (1.0 / N)
        d = (t - mu) * mask
        var = jnp.sum(d * d, axis=0, keepdims=True) * (1.0 / N)
        return (t - mu) * lax.rsqrt(var + 1e-5) * gamma + beta

    def leaky(t):
        return jnp.where(t > 0, t, 0.02 * t)

    t = jnp.dot(h, m1_ref[...], preferred_element_type=jnp.float32) + mb1_ref[...]
    t = leaky(bn(t, g1_ref[...], be1_ref[...]))
    t = jnp.dot(t, m2_ref[...], preferred_element_type=jnp.float32) + mb2_ref[...]
    t = leaky(bn(t, g2_ref[...], be2_ref[...]))
    t = jnp.dot(t, m3_ref[...], preferred_element_type=jnp.float32) + mb3_ref[...]
    m = jnp.max(t, axis=1, keepdims=True)
    lse = jnp.log(jnp.sum(jnp.exp(t - m), axis=1, keepdims=True))
    out_ref[...] = t - m - lse


_tc_head = pl.pallas_call(
    _tc_head_body,
    out_shape=jax.ShapeDtypeStruct((NPAD, C), jnp.float32),
)


def kernel(x, edge_index, W1, b1, W2, b2, W3, b3,
           M1, mb1, g1, be1, M2, mb2, g2, be2, M3, mb3):
    x = jnp.squeeze(x)
    src, dst = edge_index[0], edge_index[1]
    pad = EPAD - E
    srcp = jnp.concatenate([src, jnp.zeros((pad,), jnp.int32)]).reshape(NTILES, NB, B)
    dstp = jnp.concatenate([dst, jnp.full((pad,), N, jnp.int32)]).reshape(NTILES, NB, B)

    ones_col = jnp.ones((B, 1), jnp.float32)
    zeros_col = jnp.zeros((NPAD, 1), jnp.float32)
    zeros_rows = jnp.zeros((NPAD, 16), jnp.float32)

    deg_kernel, prop_kernel = _sc_kernels()
    degp = deg_kernel(dstp, ones_col, zeros_col)
    gv, dinv = _tc1(x, W1, degp)
    sp = prop_kernel(gv, srcp, dstp, zeros_rows)
    gv = _tc_mid(sp, gv, dinv, b1, W2)
    sp = prop_kernel(gv, srcp, dstp, zeros_rows)
    gv = _tc_mid(sp, gv, dinv, b2, W3)
    sp = prop_kernel(gv, srcp, dstp, zeros_rows)
    out = _tc_head(sp, gv, dinv, b3, M1, mb1, g1, be1, M2, mb2, g2, be2, M3, mb3)
    return out[:N]


# speedup vs baseline: 34.6310x; 1.0001x over previous
import functools

import jax
import jax.numpy as jnp
from jax import lax
from jax.experimental import pallas as pl
from jax.experimental.pallas import tpu as pltpu
from jax.experimental.pallas import tpu_sc as plsc

N = 10000
E = 320000
D = 128
C = 40

NC = 2
NS = 16
NTILES = NC * NS
B = 128
NB = 80
EPAD = NTILES * NB * B
NSLOT = 8
AHEAD = 4
NPAD = 10240
RPS = NPAD // NS

def _wait_copy(src, dst, sem):
    pltpu.make_async_copy(src, dst, sem).wait()


def _deg_body(dst_hbm, ones_hbm, zeros_hbm, deg_out, dstv, onesv, acc):
    c = lax.axis_index("c")
    s = lax.axis_index("s")
    wid = c * NS + s
    pltpu.sync_copy(dst_hbm.at[wid], dstv)
    pltpu.sync_copy(ones_hbm, onesv)
    pltpu.sync_copy(zeros_hbm.at[pl.ds(s * RPS, RPS)], acc.at[pl.ds(s * RPS, RPS)])
    plsc.subcore_barrier()

    def step(j, carry):
        pltpu.sync_copy(onesv, acc.at[dstv.at[j]], add=True)
        return carry

    lax.fori_loop(0, NB, step, 0)
    plsc.subcore_barrier()
    pltpu.sync_copy(acc.at[pl.ds(s * RPS, RPS)],
                    deg_out.at[c, pl.ds(s * RPS, RPS)])


def _prop_body(g_hbm, src_hbm, dst_hbm, zeros_hbm, out_hbm,
               srcv, dstv, bufs, acc, *gsems):
    c = lax.axis_index("c")
    s = lax.axis_index("s")
    wid = c * NS + s
    pltpu.sync_copy(src_hbm.at[wid], srcv)
    pltpu.sync_copy(dst_hbm.at[wid], dstv)
    pltpu.sync_copy(zeros_hbm.at[pl.ds(s * RPS, RPS)], acc.at[pl.ds(s * RPS, RPS)])
    plsc.subcore_barrier()

    def fire_gather(j, slot):
        pltpu.async_copy(g_hbm.at[srcv.at[j]], bufs[slot], gsems[slot])

    def wait_gather(slot):
        _wait_copy(g_hbm.at[srcv.at[0]], bufs[slot], gsems[slot])

    for j in range(NSLOT):
        fire_gather(j, j)

    def group(g, carry):
        base = g * NSLOT
        for b in range(NSLOT):
            i = base + b
            wait_gather(b)
            pltpu.sync_copy(bufs[b], acc.at[dstv.at[i]], add=True)
            fire_gather(i + NSLOT, b)
        return carry

    lax.fori_loop(0, NB // NSLOT - 1, group, 0)

    for i in range(NB - NSLOT, NB):
        b = i % NSLOT
        wait_gather(b)
        pltpu.sync_copy(bufs[b], acc.at[dstv.at[i]], add=True)

    plsc.subcore_barrier()
    pltpu.sync_copy(acc.at[pl.ds(s * RPS, RPS)],
                    out_hbm.at[c, pl.ds(s * RPS, RPS)])


@functools.cache
def _sc_kernels():
    mesh = plsc.VectorSubcoreMesh(core_axis_name="c", subcore_axis_name="s",
                                  num_cores=NC, num_subcores=NS)
    params = pltpu.CompilerParams(use_tc_tiling_on_sc=False)
    deg = pl.kernel(
        _deg_body,
        out_type=jax.ShapeDtypeStruct((NC, NPAD, 1), jnp.float32),
        mesh=mesh,
        compiler_params=params,
        scratch_types=[
            pltpu.VMEM((NB, B), jnp.int32),
            pltpu.VMEM((B, 1), jnp.float32),
            pltpu.VMEM_SHARED((NPAD, 1), jnp.float32),
        ],
    )
    prop = pl.kernel(
        _prop_body,
        out_type=jax.ShapeDtypeStruct((NC, NPAD, 16), jnp.float32),
        mesh=mesh,
        compiler_params=params,
        scratch_types=[
            pltpu.VMEM((NB, B), jnp.int32),
            pltpu.VMEM((NB, B), jnp.int32),
            tuple(pltpu.VMEM((B, 16), jnp.float32) for _ in range(NSLOT)),
            pltpu.VMEM_SHARED((NPAD, 16), jnp.float32),
        ] + [pltpu.SemaphoreType.DMA for _ in range(NSLOT)],
    )
    return deg, prop


def _tc1_body(x_ref, w_ref, degp_ref, g_ref, dinv_ref):
    deg = degp_ref[0] + degp_ref[1] + 1.0
    dinv = lax.rsqrt(deg)
    dinv_ref[...] = dinv
    h = jnp.dot(x_ref[...], w_ref[...], preferred_element_type=jnp.float32)
    g_ref[0:N] = h * dinv[0:N]
    g_ref[N:NPAD] = jnp.zeros((NPAD - N, 16), jnp.float32)


_tc1 = pl.pallas_call(
    _tc1_body,
    out_shape=(jax.ShapeDtypeStruct((NPAD, 16), jnp.float32),
               jax.ShapeDtypeStruct((NPAD, 1), jnp.float32)),
)


def _tc_mid_body(sp_ref, g_ref, dinv_ref, b_ref, w_ref, gout_ref):
    dinv = dinv_ref[...]
    h = dinv * (sp_ref[0] + sp_ref[1] + g_ref[...]) + b_ref[...]
    h = jnp.maximum(h, 0.0)
    gout_ref[...] = jnp.dot(h, w_ref[...], preferred_element_type=jnp.float32) * dinv


_tc_mid = pl.pallas_call(
    _tc_mid_body,
    out_shape=jax.ShapeDtypeStruct((NPAD, 16), jnp.float32),
)


def _tc_head_body(sp_ref, g_ref, dinv_ref, b3_ref, m1_ref, mb1_ref, g1_ref,
                  be1_ref, m2_ref, mb2_ref, g2_ref, be2_ref, m3_ref, mb3_ref,
                  out_ref):
    h = dinv_ref[...] * (sp_ref[0] + sp_ref[1] + g_ref[...]) + b3_ref[...]
    mask = (lax.broadcasted_iota(jnp.int32, (NPAD, 1), 0) < N).astype(jnp.float32)

    def bn(t, gamma, beta):
        mu = jnp.sum(t * mask, axis=0, keepdims=True) * (1.0 / N)
        d = (t - mu) * mask
        var = jnp.sum(d * d, axis=0, keepdims=True) * (1.0 / N)
        return (t - mu) * lax.rsqrt(var + 1e-5) * gamma + beta

    def leaky(t):
        return jnp.where(t > 0, t, 0.02 * t)

    t = jnp.dot(h, m1_ref[...], preferred_element_type=jnp.float32) + mb1_ref[...]
    t = leaky(bn(t, g1_ref[...], be1_ref[...]))
    t = jnp.dot(t, m2_ref[...], preferred_element_type=jnp.float32) + mb2_ref[...]
    t = leaky(bn(t, g2_ref[...], be2_ref[...]))
    t = jnp.dot(t, m3_ref[...], preferred_element_type=jnp.float32) + mb3_ref[...]
    m = jnp.max(t, axis=1, keepdims=True)
    lse = jnp.log(jnp.sum(jnp.exp(t - m), axis=1, keepdims=True))
    out_ref[...] = t - m - lse


_tc_head = pl.pallas_call(
    _tc_head_body,
    out_shape=jax.ShapeDtypeStruct((NPAD, C), jnp.float32),
)


def kernel(x, edge_index, W1, b1, W2, b2, W3, b3,
           M1, mb1, g1, be1, M2, mb2, g2, be2, M3, mb3):
    x = jnp.squeeze(x)
    src, dst = edge_index[0], edge_index[1]
    pad = EPAD - E
    srcp = jnp.concatenate([src, jnp.zeros((pad,), jnp.int32)]).reshape(NTILES, NB, B)
    dstp = jnp.concatenate([dst, jnp.full((pad,), N, jnp.int32)]).reshape(NTILES, NB, B)

    ones_col = jnp.ones((B, 1), jnp.float32)
    zeros_col = jnp.zeros((NPAD, 1), jnp.float32)
    zeros_rows = jnp.zeros((NPAD, 16), jnp.float32)

    deg_kernel, prop_kernel = _sc_kernels()
    degp = deg_kernel(dstp, ones_col, zeros_col)
    gv, dinv = _tc1(x, W1, degp)
    sp = prop_kernel(gv, srcp, dstp, zeros_rows)
    gv = _tc_mid(sp, gv, dinv, b1, W2)
    sp = prop_kernel(gv, srcp, dstp, zeros_rows)
    gv = _tc_mid(sp, gv, dinv, b2, W3)
    sp = prop_kernel(gv, srcp, dstp, zeros_rows)
    out = _tc_head(sp, gv, dinv, b3, M1, mb1, g1, be1, M2, mb2, g2, be2, M3, mb3)
    return out[:N]


# speedup vs baseline: 49.3225x; 1.4242x over previous
import functools

import jax
import jax.numpy as jnp
from jax import lax
from jax.experimental import pallas as pl
from jax.experimental.pallas import tpu as pltpu
from jax.experimental.pallas import tpu_sc as plsc

N = 10000
E = 320000
D = 128
C = 40

NC = 2
NS = 16
NTILES = NC * NS
B = 128
NB = 80
EPAD = NTILES * NB * B
NSLOT = 8
AHEAD = 4
NPAD = 10240
RPS = NPAD // NS

def _wait_copy(src, dst, sem):
    pltpu.make_async_copy(src, dst, sem).wait()


def _deg_body(dst_hbm, ones_hbm, zeros_hbm, deg_out, dstv, onesv, acc):
    c = lax.axis_index("c")
    s = lax.axis_index("s")
    wid = c * NS + s
    pltpu.sync_copy(dst_hbm.at[wid], dstv)
    pltpu.sync_copy(ones_hbm, onesv)
    pltpu.sync_copy(zeros_hbm.at[pl.ds(s * RPS, RPS)], acc.at[pl.ds(s * RPS, RPS)])
    plsc.subcore_barrier()

    def step(j, carry):
        pltpu.sync_copy(onesv, acc.at[dstv.at[j]], add=True)
        return carry

    lax.fori_loop(0, NB, step, 0)
    plsc.subcore_barrier()
    pltpu.sync_copy(acc.at[pl.ds(s * RPS, RPS)],
                    deg_out.at[c, pl.ds(s * RPS, RPS)])


def _prop_body(g_hbm, src_hbm, dst_hbm, zeros_hbm, out_hbm,
               srcv, dstv, bufs, gspm, acc, *gsems):
    c = lax.axis_index("c")
    s = lax.axis_index("s")
    wid = c * NS + s
    pltpu.sync_copy(src_hbm.at[wid], srcv)
    pltpu.sync_copy(dst_hbm.at[wid], dstv)
    pltpu.sync_copy(g_hbm.at[pl.ds(s * RPS, RPS)], gspm.at[pl.ds(s * RPS, RPS)])
    pltpu.sync_copy(zeros_hbm.at[pl.ds(s * RPS, RPS)], acc.at[pl.ds(s * RPS, RPS)])
    plsc.subcore_barrier()

    def fire_gather(j, slot):
        pltpu.async_copy(gspm.at[srcv.at[j]], bufs[slot], gsems[slot])

    def wait_gather(slot):
        _wait_copy(gspm.at[srcv.at[0]], bufs[slot], gsems[slot])

    for j in range(NSLOT):
        fire_gather(j, j)

    def group(g, carry):
        base = g * NSLOT
        for b in range(NSLOT):
            i = base + b
            wait_gather(b)
            pltpu.sync_copy(bufs[b], acc.at[dstv.at[i]], add=True)
            fire_gather(i + NSLOT, b)
        return carry

    lax.fori_loop(0, NB // NSLOT - 1, group, 0)

    for i in range(NB - NSLOT, NB):
        b = i % NSLOT
        wait_gather(b)
        pltpu.sync_copy(bufs[b], acc.at[dstv.at[i]], add=True)

    plsc.subcore_barrier()
    pltpu.sync_copy(acc.at[pl.ds(s * RPS, RPS)],
                    out_hbm.at[c, pl.ds(s * RPS, RPS)])


@functools.cache
def _sc_kernels():
    mesh = plsc.VectorSubcoreMesh(core_axis_name="c", subcore_axis_name="s",
                                  num_cores=NC, num_subcores=NS)
    params = pltpu.CompilerParams(use_tc_tiling_on_sc=False)
    deg = pl.kernel(
        _deg_body,
        out_type=jax.ShapeDtypeStruct((NC, NPAD, 1), jnp.float32),
        mesh=mesh,
        compiler_params=params,
        scratch_types=[
            pltpu.VMEM((NB, B), jnp.int32),
            pltpu.VMEM((B, 1), jnp.float32),
            pltpu.VMEM_SHARED((NPAD, 1), jnp.float32),
        ],
    )
    prop = pl.kernel(
        _prop_body,
        out_type=jax.ShapeDtypeStruct((NC, NPAD, 16), jnp.float32),
        mesh=mesh,
        compiler_params=params,
        scratch_types=[
            pltpu.VMEM((NB, B), jnp.int32),
            pltpu.VMEM((NB, B), jnp.int32),
            tuple(pltpu.VMEM((B, 16), jnp.float32) for _ in range(NSLOT)),
            pltpu.VMEM_SHARED((NPAD, 16), jnp.float32),
            pltpu.VMEM_SHARED((NPAD, 16), jnp.float32),
        ] + [pltpu.SemaphoreType.DMA for _ in range(NSLOT)],
    )
    return deg, prop


def _tc1_body(x_ref, w_ref, degp_ref, g_ref, dinv_ref):
    deg = degp_ref[0] + degp_ref[1] + 1.0
    dinv = lax.rsqrt(deg)
    dinv_ref[...] = dinv
    h = jnp.dot(x_ref[...], w_ref[...], preferred_element_type=jnp.float32)
    g_ref[0:N] = h * dinv[0:N]
    g_ref[N:NPAD] = jnp.zeros((NPAD - N, 16), jnp.float32)


_tc1 = pl.pallas_call(
    _tc1_body,
    out_shape=(jax.ShapeDtypeStruct((NPAD, 16), jnp.float32),
               jax.ShapeDtypeStruct((NPAD, 1), jnp.float32)),
)


def _tc_mid_body(sp_ref, g_ref, dinv_ref, b_ref, w_ref, gout_ref):
    dinv = dinv_ref[...]
    h = dinv * (sp_ref[0] + sp_ref[1] + g_ref[...]) + b_ref[...]
    h = jnp.maximum(h, 0.0)
    gout_ref[...] = jnp.dot(h, w_ref[...], preferred_element_type=jnp.float32) * dinv


_tc_mid = pl.pallas_call(
    _tc_mid_body,
    out_shape=jax.ShapeDtypeStruct((NPAD, 16), jnp.float32),
)


def _tc_head_body(sp_ref, g_ref, dinv_ref, b3_ref, m1_ref, mb1_ref, g1_ref,
                  be1_ref, m2_ref, mb2_ref, g2_ref, be2_ref, m3_ref, mb3_ref,
                  out_ref):
    h = dinv_ref[...] * (sp_ref[0] + sp_ref[1] + g_ref[...]) + b3_ref[...]
    mask = (lax.broadcasted_iota(jnp.int32, (NPAD, 1), 0) < N).astype(jnp.float32)

    def bn(t, gamma, beta):
        mu = jnp.sum(t * mask, axis=0, keepdims=True) * (1.0 / N)
        d = (t - mu) * mask
        var = jnp.sum(d * d, axis=0, keepdims=True) * (1.0 / N)
        return (t - mu) * lax.rsqrt(var + 1e-5) * gamma + beta

    def leaky(t):
        return jnp.where(t > 0, t, 0.02 * t)

    t = jnp.dot(h, m1_ref[...], preferred_element_type=jnp.float32) + mb1_ref[...]
    t = leaky(bn(t, g1_ref[...], be1_ref[...]))
    t = jnp.dot(t, m2_ref[...], preferred_element_type=jnp.float32) + mb2_ref[...]
    t = leaky(bn(t, g2_ref[...], be2_ref[...]))
    t = jnp.dot(t, m3_ref[...], preferred_element_type=jnp.float32) + mb3_ref[...]
    m = jnp.max(t, axis=1, keepdims=True)
    lse = jnp.log(jnp.sum(jnp.exp(t - m), axis=1, keepdims=True))
    out_ref[...] = t - m - lse


_tc_head = pl.pallas_call(
    _tc_head_body,
    out_shape=jax.ShapeDtypeStruct((NPAD, C), jnp.float32),
)


def kernel(x, edge_index, W1, b1, W2, b2, W3, b3,
           M1, mb1, g1, be1, M2, mb2, g2, be2, M3, mb3):
    x = jnp.squeeze(x)
    src, dst = edge_index[0], edge_index[1]
    pad = EPAD - E
    srcp = jnp.concatenate([src, jnp.zeros((pad,), jnp.int32)]).reshape(NTILES, NB, B)
    dstp = jnp.concatenate([dst, jnp.full((pad,), N, jnp.int32)]).reshape(NTILES, NB, B)

    ones_col = jnp.ones((B, 1), jnp.float32)
    zeros_col = jnp.zeros((NPAD, 1), jnp.float32)
    zeros_rows = jnp.zeros((NPAD, 16), jnp.float32)

    deg_kernel, prop_kernel = _sc_kernels()
    degp = deg_kernel(dstp, ones_col, zeros_col)
    gv, dinv = _tc1(x, W1, degp)
    sp = prop_kernel(gv, srcp, dstp, zeros_rows)
    gv = _tc_mid(sp, gv, dinv, b1, W2)
    sp = prop_kernel(gv, srcp, dstp, zeros_rows)
    gv = _tc_mid(sp, gv, dinv, b2, W3)
    sp = prop_kernel(gv, srcp, dstp, zeros_rows)
    out = _tc_head(sp, gv, dinv, b3, M1, mb1, g1, be1, M2, mb2, g2, be2, M3, mb3)
    return out[:N]


# speedup vs baseline: 68.7638x; 1.3942x over previous
import functools

import jax
import jax.numpy as jnp
from jax import lax
from jax.experimental import pallas as pl
from jax.experimental.pallas import tpu as pltpu
from jax.experimental.pallas import tpu_sc as plsc

N = 10000
E = 320000
D = 128
C = 40

NC = 2
NS = 16
NTILES = NC * NS
B = 128
NB = 80
EPAD = NTILES * NB * B
NSLOT = 8
AHEAD = 4
NPAD = 10240
RPS = NPAD // NS
PR = NPAD // 8
NVR = N // 8

def _wait_copy(src, dst, sem):
    pltpu.make_async_copy(src, dst, sem).wait()


def _deg_body(dst_hbm, ones_hbm, zeros_hbm, deg_out, dstv, onesv, acc):
    c = lax.axis_index("c")
    s = lax.axis_index("s")
    wid = c * NS + s
    pltpu.sync_copy(dst_hbm.at[wid], dstv)
    pltpu.sync_copy(ones_hbm, onesv)
    pltpu.sync_copy(zeros_hbm.at[pl.ds(s * RPS, RPS)], acc.at[pl.ds(s * RPS, RPS)])
    plsc.subcore_barrier()

    def step(j, carry):
        pltpu.sync_copy(onesv, acc.at[dstv.at[j]], add=True)
        return carry

    lax.fori_loop(0, NB, step, 0)
    plsc.subcore_barrier()
    pltpu.sync_copy(acc.at[pl.ds(s * RPS, RPS)],
                    deg_out.at[c, pl.ds(s * RPS, RPS)])


def _prop_body(g_hbm, src_hbm, dst_hbm, zeros_hbm, out_hbm,
               srcv, dstv, bufs, gspm, acc, *gsems):
    c = lax.axis_index("c")
    s = lax.axis_index("s")
    wid = c * NS + s
    pltpu.sync_copy(src_hbm.at[wid], srcv)
    pltpu.sync_copy(dst_hbm.at[wid], dstv)
    pltpu.sync_copy(g_hbm.at[pl.ds(s * RPS, RPS)], gspm.at[pl.ds(s * RPS, RPS)])
    pltpu.sync_copy(zeros_hbm.at[pl.ds(s * RPS, RPS)], acc.at[pl.ds(s * RPS, RPS)])
    plsc.subcore_barrier()

    def fire_gather(j, slot):
        pltpu.async_copy(gspm.at[srcv.at[j]], bufs[slot], gsems[slot])

    def wait_gather(slot):
        _wait_copy(gspm.at[srcv.at[0]], bufs[slot], gsems[slot])

    for j in range(NSLOT):
        fire_gather(j, j)

    def group(g, carry):
        base = g * NSLOT
        for b in range(NSLOT):
            i = base + b
            wait_gather(b)
            pltpu.sync_copy(bufs[b], acc.at[dstv.at[i]], add=True)
            fire_gather(i + NSLOT, b)
        return carry

    lax.fori_loop(0, NB // NSLOT - 1, group, 0)

    for i in range(NB - NSLOT, NB):
        b = i % NSLOT
        wait_gather(b)
        pltpu.sync_copy(bufs[b], acc.at[dstv.at[i]], add=True)

    plsc.subcore_barrier()
    pltpu.sync_copy(acc.at[pl.ds(s * RPS, RPS)],
                    out_hbm.at[c, pl.ds(s * RPS, RPS)])


@functools.cache
def _sc_kernels():
    mesh = plsc.VectorSubcoreMesh(core_axis_name="c", subcore_axis_name="s",
                                  num_cores=NC, num_subcores=NS)
    params = pltpu.CompilerParams(use_tc_tiling_on_sc=False)
    deg = pl.kernel(
        _deg_body,
        out_type=jax.ShapeDtypeStruct((NC, NPAD, 16), jnp.float32),
        mesh=mesh,
        compiler_params=params,
        scratch_types=[
            pltpu.VMEM((NB, B), jnp.int32),
            pltpu.VMEM((B, 16), jnp.float32),
            pltpu.VMEM_SHARED((NPAD, 16), jnp.float32),
        ],
    )
    prop = pl.kernel(
        _prop_body,
        out_type=jax.ShapeDtypeStruct((NC, NPAD, 16), jnp.float32),
        mesh=mesh,
        compiler_params=params,
        scratch_types=[
            pltpu.VMEM((NB, B), jnp.int32),
            pltpu.VMEM((NB, B), jnp.int32),
            tuple(pltpu.VMEM((B, 16), jnp.float32) for _ in range(NSLOT)),
            pltpu.VMEM_SHARED((NPAD, 16), jnp.float32),
            pltpu.VMEM_SHARED((NPAD, 16), jnp.float32),
        ] + [pltpu.SemaphoreType.DMA for _ in range(NSLOT)],
    )
    return deg, prop


def _tc1_body(xp_ref, w_ref, degp_ref, g_ref, dinv_ref):
    deg = degp_ref[0] + degp_ref[1] + 1.0
    dinv = lax.rsqrt(deg)
    dinv_ref[...] = dinv
    h = jnp.dot(xp_ref[...], w_ref[...], preferred_element_type=jnp.float32)
    g_ref[...] = h * dinv


_tc1 = pl.pallas_call(
    _tc1_body,
    out_shape=(jax.ShapeDtypeStruct((PR, 128), jnp.float32),
               jax.ShapeDtypeStruct((PR, 128), jnp.float32)),
)


def _tc_mid_body(sp_ref, g_ref, dinv_ref, b_ref, w_ref, gout_ref):
    dinv = dinv_ref[...]
    h = dinv * (sp_ref[0] + sp_ref[1] + g_ref[...]) + b_ref[...]
    h = jnp.maximum(h, 0.0)
    gout_ref[...] = jnp.dot(h, w_ref[...], preferred_element_type=jnp.float32) * dinv


_tc_mid = pl.pallas_call(
    _tc_mid_body,
    out_shape=jax.ShapeDtypeStruct((PR, 128), jnp.float32),
)


def _tc_head_body(sp_ref, g_ref, dinv_ref, b3_ref, m1_ref, mb1_ref, g1_ref,
                  be1_ref, m2_ref, mb2_ref, g2_ref, be2_ref, m3_ref, mb3_ref,
                  out_ref):
    h = dinv_ref[...] * (sp_ref[0] + sp_ref[1] + g_ref[...]) + b3_ref[...]
    mask = (lax.broadcasted_iota(jnp.int32, (PR, 1), 0) < NVR).astype(jnp.float32)

    def slotsum(v, f):
        r = v[0:f]
        for k in range(1, 8):
            r = r + v[k * f:(k + 1) * f]
        return r

    def bn(t, gamma, beta, f):
        mu = jnp.tile(slotsum(jnp.sum(t * mask, axis=0), f) * (1.0 / N), 8)
        d = (t - mu) * mask
        var = jnp.tile(slotsum(jnp.sum(d * d, axis=0), f) * (1.0 / N), 8)
        return (t - mu) * lax.rsqrt(var + 1e-5) * gamma + beta

    def leaky(t):
        return jnp.where(t > 0, t, 0.02 * t)

    t = jnp.dot(h, m1_ref[...], preferred_element_type=jnp.float32) + mb1_ref[...]
    t = leaky(bn(t, g1_ref[...], be1_ref[...], 64))
    t = jnp.dot(t, m2_ref[...], preferred_element_type=jnp.float32) + mb2_ref[...]
    t = leaky(bn(t, g2_ref[...], be2_ref[...], 16))
    t = jnp.dot(t, m3_ref[...], preferred_element_type=jnp.float32) + mb3_ref[...]
    outs = []
    for k in range(8):
        ck = t[:, k * C:(k + 1) * C]
        m = jnp.max(ck, axis=1, keepdims=True)
        lse = jnp.log(jnp.sum(jnp.exp(ck - m), axis=1, keepdims=True))
        outs.append(ck - m - lse)
    out_ref[...] = jnp.concatenate(outs, axis=1)


_tc_head = pl.pallas_call(
    _tc_head_body,
    out_shape=jax.ShapeDtypeStruct((PR, 8 * C), jnp.float32),
)


def _bd(w):
    return jnp.kron(jnp.eye(8, dtype=jnp.float32), w)


def kernel(x, edge_index, W1, b1, W2, b2, W3, b3,
           M1, mb1, g1, be1, M2, mb2, g2, be2, M3, mb3):
    x = jnp.squeeze(x)
    src, dst = edge_index[0], edge_index[1]
    pad = EPAD - E
    srcp = jnp.concatenate([src, jnp.zeros((pad,), jnp.int32)]).reshape(NTILES, NB, B)
    dstp = jnp.concatenate([dst, jnp.full((pad,), N, jnp.int32)]).reshape(NTILES, NB, B)

    ones_rows = jnp.ones((B, 16), jnp.float32)
    zeros_rows = jnp.zeros((NPAD, 16), jnp.float32)
    xp = jnp.pad(x, ((0, NPAD - N), (0, 0))).reshape(PR, 8 * D)

    deg_kernel, prop_kernel = _sc_kernels()
    degp = deg_kernel(dstp, ones_rows, zeros_rows).reshape(NC, PR, 128)
    gv, dinv = _tc1(xp, _bd(W1), degp)
    sp = prop_kernel(gv.reshape(NPAD, 16), srcp, dstp, zeros_rows).reshape(NC, PR, 128)
    gv = _tc_mid(sp, gv, dinv, jnp.tile(b1, 8), _bd(W2))
    sp = prop_kernel(gv.reshape(NPAD, 16), srcp, dstp, zeros_rows).reshape(NC, PR, 128)
    gv = _tc_mid(sp, gv, dinv, jnp.tile(b2, 8), _bd(W3))
    sp = prop_kernel(gv.reshape(NPAD, 16), srcp, dstp, zeros_rows).reshape(NC, PR, 128)
    out = _tc_head(sp, gv, dinv, jnp.tile(b3, 8), _bd(M1), jnp.tile(mb1, 8),
                   jnp.tile(g1, 8), jnp.tile(be1, 8), _bd(M2), jnp.tile(mb2, 8),
                   jnp.tile(g2, 8), jnp.tile(be2, 8), _bd(M3), jnp.tile(mb3, 8))
    return out.reshape(NPAD, C)[:N]


# speedup vs baseline: 74.1878x; 1.0789x over previous
import functools

import jax
import jax.numpy as jnp
from jax import lax
from jax.experimental import pallas as pl
from jax.experimental.pallas import tpu as pltpu
from jax.experimental.pallas import tpu_sc as plsc

N = 10000
E = 320000
D = 128
C = 40

NC = 2
NS = 16
NTILES = NC * NS
B = 128
NB = 80
EPAD = NTILES * NB * B
NSLOT = 8
AHEAD = 4
NPAD = 10240
RPS = NPAD // NS
PR = NPAD // 8
NVR = N // 8

def _wait_copy(src, dst, sem):
    pltpu.make_async_copy(src, dst, sem).wait()


def _deg_body(edges_hbm, ones_hbm, zeros_hbm, deg_out, dstv, onesv, acc):
    c = lax.axis_index("c")
    s = lax.axis_index("s")
    wid = c * NS + s
    pltpu.sync_copy(edges_hbm.at[1, wid], dstv)
    pltpu.sync_copy(ones_hbm, onesv)
    pltpu.sync_copy(zeros_hbm.at[pl.ds(s * RPS, RPS)], acc.at[pl.ds(s * RPS, RPS)])
    plsc.subcore_barrier()

    def step(j, carry):
        pltpu.sync_copy(onesv, acc.at[dstv.at[j]], add=True)
        return carry

    lax.fori_loop(0, NB, step, 0)
    plsc.subcore_barrier()
    pltpu.sync_copy(acc.at[pl.ds(s * RPS, RPS)],
                    deg_out.at[c, pl.ds(s * RPS, RPS)])


def _prop_body(g_hbm, edges_hbm, zeros_hbm, out_hbm,
               srcv, dstv, bufs, gspm, acc, *gsems):
    c = lax.axis_index("c")
    s = lax.axis_index("s")
    wid = c * NS + s
    pltpu.sync_copy(edges_hbm.at[0, wid], srcv)
    pltpu.sync_copy(edges_hbm.at[1, wid], dstv)
    pltpu.sync_copy(g_hbm.at[pl.ds(s * RPS, RPS)], gspm.at[pl.ds(s * RPS, RPS)])
    pltpu.sync_copy(zeros_hbm.at[pl.ds(s * RPS, RPS)], acc.at[pl.ds(s * RPS, RPS)])
    plsc.subcore_barrier()

    def fire_gather(j, slot):
        pltpu.async_copy(gspm.at[srcv.at[j]], bufs[slot], gsems[slot])

    def wait_gather(slot):
        _wait_copy(gspm.at[srcv.at[0]], bufs[slot], gsems[slot])

    for j in range(NSLOT):
        fire_gather(j, j)

    def group(g, carry):
        base = g * NSLOT
        for b in range(NSLOT):
            i = base + b
            wait_gather(b)
            pltpu.sync_copy(bufs[b], acc.at[dstv.at[i]], add=True)
            fire_gather(i + NSLOT, b)
        return carry

    lax.fori_loop(0, NB // NSLOT - 1, group, 0)

    for i in range(NB - NSLOT, NB):
        b = i % NSLOT
        wait_gather(b)
        pltpu.sync_copy(bufs[b], acc.at[dstv.at[i]], add=True)

    plsc.subcore_barrier()
    pltpu.sync_copy(acc.at[pl.ds(s * RPS, RPS)],
                    out_hbm.at[c, pl.ds(s * RPS, RPS)])


@functools.cache
def _sc_kernels():
    mesh = plsc.VectorSubcoreMesh(core_axis_name="c", subcore_axis_name="s",
                                  num_cores=NC, num_subcores=NS)
    params = pltpu.CompilerParams(use_tc_tiling_on_sc=False)
    deg = pl.kernel(
        _deg_body,
        out_type=jax.ShapeDtypeStruct((NC, NPAD, 16), jnp.float32),
        mesh=mesh,
        compiler_params=params,
        scratch_types=[
            pltpu.VMEM((NB, B), jnp.int32),
            pltpu.VMEM((B, 16), jnp.float32),
            pltpu.VMEM_SHARED((NPAD, 16), jnp.float32),
        ],
    )
    prop = pl.kernel(
        _prop_body,
        out_type=jax.ShapeDtypeStruct((NC, NPAD, 16), jnp.float32),
        mesh=mesh,
        compiler_params=params,
        scratch_types=[
            pltpu.VMEM((NB, B), jnp.int32),
            pltpu.VMEM((NB, B), jnp.int32),
            tuple(pltpu.VMEM((B, 16), jnp.float32) for _ in range(NSLOT)),
            pltpu.VMEM_SHARED((NPAD, 16), jnp.float32),
            pltpu.VMEM_SHARED((NPAD, 16), jnp.float32),
        ] + [pltpu.SemaphoreType.DMA for _ in range(NSLOT)],
    )
    return deg, prop


def _tc0_body(xp_ref, w_ref, u_ref):
    u_ref[...] = jnp.dot(xp_ref[...], w_ref[...],
                         preferred_element_type=jnp.float32)


_tc0 = pl.pallas_call(
    _tc0_body, out_shape=jax.ShapeDtypeStruct((PR, 128), jnp.float32))


def _tc1_body(u_ref, degp_ref, g_ref, dinv_ref):
    deg = degp_ref[0] + degp_ref[1] + 1.0
    dinv = lax.rsqrt(deg)
    dinv_ref[...] = dinv
    g_ref[...] = u_ref[...] * dinv


_tc1 = pl.pallas_call(
    _tc1_body,
    out_shape=(jax.ShapeDtypeStruct((PR, 128), jnp.float32),
               jax.ShapeDtypeStruct((PR, 128), jnp.float32)),
)


def _tc_mid_body(sp_ref, g_ref, dinv_ref, b_ref, w_ref, gout_ref):
    dinv = dinv_ref[...]
    h = dinv * (sp_ref[0] + sp_ref[1] + g_ref[...]) + b_ref[...]
    h = jnp.maximum(h, 0.0)
    gout_ref[...] = jnp.dot(h, w_ref[...], preferred_element_type=jnp.float32) * dinv


_tc_mid = pl.pallas_call(
    _tc_mid_body,
    out_shape=jax.ShapeDtypeStruct((PR, 128), jnp.float32),
)


def _tc_head_body(sp_ref, g_ref, dinv_ref, b3_ref, m1_ref, mb1_ref, g1_ref,
                  be1_ref, m2_ref, mb2_ref, g2_ref, be2_ref, m3_ref, mb3_ref,
                  out_ref):
    h = dinv_ref[...] * (sp_ref[0] + sp_ref[1] + g_ref[...]) + b3_ref[...]
    mask = (lax.broadcasted_iota(jnp.int32, (PR, 1), 0) < NVR).astype(jnp.float32)

    def slotsum(v, f):
        r = v[0:f]
        for k in range(1, 8):
            r = r + v[k * f:(k + 1) * f]
        return r

    def bn(t, gamma, beta, f):
        mu = jnp.tile(slotsum(jnp.sum(t * mask, axis=0), f) * (1.0 / N), 8)
        d = (t - mu) * mask
        var = jnp.tile(slotsum(jnp.sum(d * d, axis=0), f) * (1.0 / N), 8)
        return (t - mu) * lax.rsqrt(var + 1e-5) * gamma + beta

    def leaky(t):
        return jnp.where(t > 0, t, 0.02 * t)

    t = jnp.dot(h, m1_ref[...], preferred_element_type=jnp.float32) + mb1_ref[...]
    t = leaky(bn(t, g1_ref[...], be1_ref[...], 64))
    t = jnp.dot(t, m2_ref[...], preferred_element_type=jnp.float32) + mb2_ref[...]
    t = leaky(bn(t, g2_ref[...], be2_ref[...], 16))
    t = jnp.dot(t, m3_ref[...], preferred_element_type=jnp.float32) + mb3_ref[...]
    outs = []
    for k in range(8):
        ck = t[:, k * C:(k + 1) * C]
        m = jnp.max(ck, axis=1, keepdims=True)
        lse = jnp.log(jnp.sum(jnp.exp(ck - m), axis=1, keepdims=True))
        outs.append(ck - m - lse)
    out_ref[...] = jnp.concatenate(outs, axis=1)


_tc_head = pl.pallas_call(
    _tc_head_body,
    out_shape=jax.ShapeDtypeStruct((PR, 8 * C), jnp.float32),
)


def _bd(w):
    return jnp.kron(jnp.eye(8, dtype=jnp.float32), w)


def kernel(x, edge_index, W1, b1, W2, b2, W3, b3,
           M1, mb1, g1, be1, M2, mb2, g2, be2, M3, mb3):
    x = jnp.squeeze(x)
    ep = jnp.pad(edge_index, ((0, 0), (0, EPAD - E)),
                 constant_values=N).reshape(2, NTILES, NB, B)

    ones_rows = jnp.ones((B, 16), jnp.float32)
    zeros_rows = jnp.zeros((NPAD, 16), jnp.float32)
    xp = jnp.pad(x, ((0, NPAD - N), (0, 0))).reshape(PR, 8 * D)

    deg_kernel, prop_kernel = _sc_kernels()
    uv = _tc0(xp, _bd(W1))
    degp = deg_kernel(ep, ones_rows, zeros_rows).reshape(NC, PR, 128)
    gv, dinv = _tc1(uv, degp)
    sp = prop_kernel(gv.reshape(NPAD, 16), ep, zeros_rows).reshape(NC, PR, 128)
    gv = _tc_mid(sp, gv, dinv, jnp.tile(b1, 8), _bd(W2))
    sp = prop_kernel(gv.reshape(NPAD, 16), ep, zeros_rows).reshape(NC, PR, 128)
    gv = _tc_mid(sp, gv, dinv, jnp.tile(b2, 8), _bd(W3))
    sp = prop_kernel(gv.reshape(NPAD, 16), ep, zeros_rows).reshape(NC, PR, 128)
    out = _tc_head(sp, gv, dinv, jnp.tile(b3, 8), _bd(M1), jnp.tile(mb1, 8),
                   jnp.tile(g1, 8), jnp.tile(be1, 8), _bd(M2), jnp.tile(mb2, 8),
                   jnp.tile(g2, 8), jnp.tile(be2, 8), _bd(M3), jnp.tile(mb3, 8))
    return out.reshape(NPAD, C)[:N]


# speedup vs baseline: 78.1704x; 1.0537x over previous
import functools

import jax
import jax.numpy as jnp
from jax import lax
from jax.experimental import pallas as pl
from jax.experimental.pallas import tpu as pltpu
from jax.experimental.pallas import tpu_sc as plsc

N = 10000
E = 320000
D = 128
C = 40

NC = 2
NS = 16
NTILES = NC * NS
B = 128
NB = 80
EPAD = NTILES * NB * B
NSLOT = 8
AHEAD = 4
NPAD = 10240
RPS = NPAD // NS
PR = NPAD // 8
NVR = N // 8

def _wait_copy(src, dst, sem):
    pltpu.make_async_copy(src, dst, sem).wait()


def _deg_body(edges_hbm, ones_hbm, zeros_hbm, deg_out, dstv, onesv, acc):
    c = lax.axis_index("c")
    s = lax.axis_index("s")
    wid = c * NS + s
    pltpu.sync_copy(edges_hbm.at[1, wid], dstv)
    pltpu.sync_copy(ones_hbm, onesv)
    pltpu.sync_copy(zeros_hbm.at[pl.ds(s * RPS, RPS)], acc.at[pl.ds(s * RPS, RPS)])
    plsc.subcore_barrier()

    def step(j, carry):
        pltpu.sync_copy(onesv, acc.at[dstv.at[j]], add=True)
        return carry

    lax.fori_loop(0, NB, step, 0)
    plsc.subcore_barrier()
    pltpu.sync_copy(acc.at[pl.ds(s * RPS, RPS)],
                    deg_out.at[c, pl.ds(s * RPS, RPS)])


def _prop_body(g_hbm, edges_hbm, zeros_hbm, out_hbm,
               srcv, dstv, bufs, gspm, acc, *gsems):
    c = lax.axis_index("c")
    s = lax.axis_index("s")
    wid = c * NS + s
    pltpu.sync_copy(edges_hbm.at[0, wid], srcv)
    pltpu.sync_copy(edges_hbm.at[1, wid], dstv)
    pltpu.sync_copy(g_hbm.at[pl.ds(s * RPS, RPS)], gspm.at[pl.ds(s * RPS, RPS)])
    pltpu.sync_copy(zeros_hbm.at[pl.ds(s * RPS, RPS)], acc.at[pl.ds(s * RPS, RPS)])
    plsc.subcore_barrier()

    def fire_gather(j, slot):
        pltpu.async_copy(gspm.at[srcv.at[j]], bufs[slot], gsems[slot])

    def wait_gather(slot):
        _wait_copy(gspm.at[srcv.at[0]], bufs[slot], gsems[slot])

    for j in range(NSLOT):
        fire_gather(j, j)

    def group(g, carry):
        base = g * NSLOT
        for b in range(NSLOT):
            i = base + b
            wait_gather(b)
            pltpu.sync_copy(bufs[b], acc.at[dstv.at[i]], add=True)
            fire_gather(i + NSLOT, b)
        return carry

    lax.fori_loop(0, NB // NSLOT - 1, group, 0)

    for i in range(NB - NSLOT, NB):
        b = i % NSLOT
        wait_gather(b)
        pltpu.sync_copy(bufs[b], acc.at[dstv.at[i]], add=True)

    plsc.subcore_barrier()
    pltpu.sync_copy(acc.at[pl.ds(s * RPS, RPS)],
                    out_hbm.at[c, pl.ds(s * RPS, RPS)])


@functools.cache
def _sc_kernels():
    mesh = plsc.VectorSubcoreMesh(core_axis_name="c", subcore_axis_name="s",
                                  num_cores=NC, num_subcores=NS)
    params = pltpu.CompilerParams(use_tc_tiling_on_sc=False)
    deg = pl.kernel(
        _deg_body,
        out_type=jax.ShapeDtypeStruct((NC, NPAD, 16), jnp.float32),
        mesh=mesh,
        compiler_params=params,
        scratch_types=[
            pltpu.VMEM((NB, B), jnp.int32),
            pltpu.VMEM((B, 16), jnp.float32),
            pltpu.VMEM_SHARED((NPAD, 16), jnp.float32),
        ],
    )
    prop = pl.kernel(
        _prop_body,
        out_type=jax.ShapeDtypeStruct((NC, NPAD, 16), jnp.float32),
        mesh=mesh,
        compiler_params=params,
        scratch_types=[
            pltpu.VMEM((NB, B), jnp.int32),
            pltpu.VMEM((NB, B), jnp.int32),
            tuple(pltpu.VMEM((B, 16), jnp.float32) for _ in range(NSLOT)),
            pltpu.VMEM_SHARED((NPAD, 16), jnp.float32),
            pltpu.VMEM_SHARED((NPAD, 16), jnp.float32),
        ] + [pltpu.SemaphoreType.DMA for _ in range(NSLOT)],
    )
    return deg, prop


def _tc0_body(xp_ref, w_ref, u_ref):
    u_ref[...] = jnp.dot(xp_ref[...], w_ref[...],
                         preferred_element_type=jnp.float32)


_tc0 = pl.pallas_call(
    _tc0_body, out_shape=jax.ShapeDtypeStruct((PR, 128), jnp.float32))


def _tc1_body(u_ref, degp_ref, g_ref, dinv_ref):
    deg = degp_ref[0] + degp_ref[1] + 1.0
    dinv = lax.rsqrt(deg)
    dinv_ref[...] = dinv
    g_ref[...] = u_ref[...] * dinv


_tc1 = pl.pallas_call(
    _tc1_body,
    out_shape=(jax.ShapeDtypeStruct((PR, 128), jnp.float32),
               jax.ShapeDtypeStruct((PR, 128), jnp.float32)),
)


def _tc_mid_body(sp_ref, g_ref, dinv_ref, b_ref, w_ref, gout_ref):
    dinv = dinv_ref[...]
    h = dinv * (sp_ref[0] + sp_ref[1] + g_ref[...]) + b_ref[...]
    h = jnp.maximum(h, 0.0)
    gout_ref[...] = jnp.dot(h, w_ref[...], preferred_element_type=jnp.float32) * dinv


_tc_mid = pl.pallas_call(
    _tc_mid_body,
    out_shape=jax.ShapeDtypeStruct((PR, 128), jnp.float32),
)


def _tc_head_body(sp_ref, g_ref, dinv_ref, b3_ref, m1_ref, mb1_ref, g1_ref,
                  be1_ref, m2_ref, mb2_ref, g2_ref, be2_ref, m3_ref, mb3_ref,
                  p64_ref, p16_ref, sblk_ref, out_ref):
    h = dinv_ref[...] * (sp_ref[0] + sp_ref[1] + g_ref[...]) + b3_ref[...]
    mask = (lax.broadcasted_iota(jnp.int32, (PR, 1), 0) < NVR).astype(jnp.float32)

    def bn(t, gamma, beta, p_ref):
        def slotsum(v):
            return jnp.dot(v[None, :], p_ref[...],
                           preferred_element_type=jnp.float32)[0]

        mu = slotsum(jnp.sum(t * mask, axis=0)) * (1.0 / N)
        d = (t - mu) * mask
        var = slotsum(jnp.sum(d * d, axis=0)) * (1.0 / N)
        return (t - mu) * lax.rsqrt(var + 1e-5) * gamma + beta

    def leaky(t):
        return jnp.where(t > 0, t, 0.02 * t)

    t = jnp.dot(h, m1_ref[...], preferred_element_type=jnp.float32) + mb1_ref[...]
    t = leaky(bn(t, g1_ref[...], be1_ref[...], p64_ref))
    t = jnp.dot(t, m2_ref[...], preferred_element_type=jnp.float32) + mb2_ref[...]
    t = leaky(bn(t, g2_ref[...], be2_ref[...], p16_ref))
    t = jnp.dot(t, m3_ref[...], preferred_element_type=jnp.float32) + mb3_ref[...]
    m = jnp.max(t, axis=1, keepdims=True)
    e = jnp.exp(t - m)
    s = jnp.dot(e, sblk_ref[...], preferred_element_type=jnp.float32)
    out_ref[...] = (t - m) - jnp.log(s)


_tc_head = pl.pallas_call(
    _tc_head_body,
    out_shape=jax.ShapeDtypeStruct((PR, 8 * C), jnp.float32),
)


def _bd(w):
    return jnp.kron(jnp.eye(8, dtype=jnp.float32), w)


def kernel(x, edge_index, W1, b1, W2, b2, W3, b3,
           M1, mb1, g1, be1, M2, mb2, g2, be2, M3, mb3):
    x = jnp.squeeze(x)
    ep = jnp.pad(edge_index, ((0, 0), (0, EPAD - E)),
                 constant_values=N).reshape(2, NTILES, NB, B)

    ones_rows = jnp.ones((B, 16), jnp.float32)
    zeros_rows = jnp.zeros((NPAD, 16), jnp.float32)
    xp = jnp.pad(x, ((0, NPAD - N), (0, 0))).reshape(PR, 8 * D)

    deg_kernel, prop_kernel = _sc_kernels()
    uv = _tc0(xp, _bd(W1))
    degp = deg_kernel(ep, ones_rows, zeros_rows).reshape(NC, PR, 128)
    gv, dinv = _tc1(uv, degp)
    sp = prop_kernel(gv.reshape(NPAD, 16), ep, zeros_rows).reshape(NC, PR, 128)
    gv = _tc_mid(sp, gv, dinv, jnp.tile(b1, 8), _bd(W2))
    sp = prop_kernel(gv.reshape(NPAD, 16), ep, zeros_rows).reshape(NC, PR, 128)
    gv = _tc_mid(sp, gv, dinv, jnp.tile(b2, 8), _bd(W3))
    sp = prop_kernel(gv.reshape(NPAD, 16), ep, zeros_rows).reshape(NC, PR, 128)
    ones8 = jnp.ones((8, 8), jnp.float32)
    p64 = jnp.kron(ones8, jnp.eye(64, dtype=jnp.float32))
    p16 = jnp.kron(ones8, jnp.eye(16, dtype=jnp.float32))
    sblk = jnp.kron(jnp.eye(8, dtype=jnp.float32), jnp.ones((C, C), jnp.float32))
    out = _tc_head(sp, gv, dinv, jnp.tile(b3, 8), _bd(M1), jnp.tile(mb1, 8),
                   jnp.tile(g1, 8), jnp.tile(be1, 8), _bd(M2), jnp.tile(mb2, 8),
                   jnp.tile(g2, 8), jnp.tile(be2, 8), _bd(M3), jnp.tile(mb3, 8),
                   p64, p16, sblk)
    return out.reshape(NPAD, C)[:N]
